# Initial kernel scaffold; baseline (speedup 1.0000x reference)
import jax
import jax.numpy as jnp
from jax.experimental import pallas as pl


def kernel(x, edge_index0, edge_index1, edge_index2, ut0_row, ut0_col, ut0_val, ut1_row, ut1_col, ut1_val, ut2_row, ut2_col, ut2_val, fc_w, fc_b, W0, b0, W1, b1, W2, b2, Wr, br):
    raise NotImplementedError("write your pallas kernel here")


# speedup vs baseline: 1.0021x; 1.0021x over previous
import jax
import jax.numpy as jnp
from jax.experimental import pallas as pl

FILTERS = [3, 16, 16, 16, 32]
NS = [10000, 2500, 625, 157]
LAMBDA_MAX = 2.0


def _cheb_j(x, row, col, W, b):
    N = x.shape[1]
    deg = jnp.zeros((N,), x.dtype).at[row].add(jnp.ones(row.shape, x.dtype))
    safe = jnp.where(deg > 0, deg, 1.0)
    dinv = jnp.where(deg > 0, 1.0 / jnp.sqrt(safe), 0.0)
    w = -(dinv[row] * dinv[col])

    def prop(h):
        msg = h[:, col, :] * w[None, :, None]
        return jnp.zeros_like(h).at[:, row, :].add(msg)

    Tx0 = x
    out = jnp.einsum('bnc,cd->bnd', Tx0, W[0])
    Tx1 = prop(Tx0)
    out = out + jnp.einsum('bnc,cd->bnd', Tx1, W[1])
    for kk in range(2, W.shape[0]):
        Tx2 = 2.0 * prop(Tx1) - Tx0
        out = out + jnp.einsum('bnc,cd->bnd', Tx2, W[kk])
        Tx0, Tx1 = Tx1, Tx2
    return out + b


def _pool_j(x, row, col, val, n_out):
    g = x[:, col, :] * val[None, :, None]
    return jnp.zeros((x.shape[0], n_out, x.shape[2]), x.dtype).at[:, row, :].add(g)


def _bn_j(x, eps=1e-5):
    m = jnp.mean(x, axis=(0, 2), keepdims=True)
    v = jnp.var(x, axis=(0, 2), keepdims=True)
    return (x - m) / jnp.sqrt(v + eps)


def _elu_pallas(x):
    B, N, C = x.shape
    xf = x.reshape(B * N, C)

    def body(x_ref, o_ref):
        v = x_ref[...]
        o_ref[...] = jnp.where(v > 0, v, jnp.exp(v) - 1.0)

    blk = 2000
    out = pl.pallas_call(
        body,
        grid=(xf.shape[0] // blk,),
        in_specs=[pl.BlockSpec((blk, C), lambda i: (i, 0))],
        out_specs=pl.BlockSpec((blk, C), lambda i: (i, 0)),
        out_shape=jax.ShapeDtypeStruct(xf.shape, xf.dtype),
    )(xf)
    return out.reshape(B, N, C)


def kernel(x, edge_index0, edge_index1, edge_index2, ut0_row, ut0_col, ut0_val, ut1_row, ut1_col, ut1_val, ut2_row, ut2_col, ut2_val, fc_w, fc_b, W0, b0, W1, b1, W2, b2, Wr, br):
    h = x @ fc_w + fc_b
    h = h.reshape(x.shape[0], NS[3], FILTERS[-1])
    h = _pool_j(h, ut2_row, ut2_col, ut2_val, NS[2])
    h = _cheb_j(h, edge_index2[0], edge_index2[1], W0, b0)
    h = _elu_pallas(_bn_j(h))
    h = _pool_j(h, ut1_row, ut1_col, ut1_val, NS[1])
    h = _cheb_j(h, edge_index1[0], edge_index1[1], W1, b1)
    h = _elu_pallas(_bn_j(h))
    h = _pool_j(h, ut0_row, ut0_col, ut0_val, NS[0])
    h = _cheb_j(h, edge_index0[0], edge_index0[1], W2, b2)
    h = _elu_pallas(_bn_j(h))
    h = _cheb_j(h, edge_index0[0], edge_index0[1], Wr, br)
    return h


# speedup vs baseline: 21.5188x; 21.4744x over previous
import functools

import jax
import jax.numpy as jnp
from jax import lax
from jax.experimental import pallas as pl
from jax.experimental.pallas import tpu as pltpu
from jax.experimental.pallas import tpu_sc as plsc

F32 = jnp.float32
I32 = jnp.int32

NBATCH = 16
N0, N1, N2, N3 = 10000, 2500, 625, 157
NC = 2
NT = 16
KG = 64


def _rup(x, m):
    return (x + m - 1) // m * m


NA0 = _rup(N0 + 1, NT * 128)
NA1 = _rup(N1 + 1, NT * 128)
NA2 = _rup(N2 + 1, NT * 128)

N_CAT = N0 + N1 + N2
NA_CAT = _rup(N_CAT + 1, NT * 8)
E_CAT = 160000 + 40000 + 10000
E_CAT_PAD = _rup(E_CAT, NT * 64 * 16)
CHD = E_CAT_PAD // (NT * 64)

_MESH = plsc.VectorSubcoreMesh(
    core_axis_name="c", subcore_axis_name="s", num_cores=NC, num_subcores=NT)


def _degscale_body(rows_hbm, deg_hbm, sh, ones_v, ridx_g):
    c = lax.axis_index("c")
    s = lax.axis_index("s")
    slab = NA_CAT // NT
    slab0 = pl.multiple_of(s * slab, 8)
    zero16 = jnp.zeros((16,), F32)
    one16 = jnp.ones((16,), F32)

    def zb(i, _):
        for dd in range(8):
            ones_v[i, pl.ds(dd * 16, 16)] = zero16
        return 0
    lax.fori_loop(0, KG, zb, 0)
    nfull = slab // KG
    rem = slab % KG

    def zc(i, _):
        pltpu.sync_copy(ones_v,
                        sh.at[pl.ds(pl.multiple_of(slab0 + i * KG, 8), KG)])
        return 0
    lax.fori_loop(0, nfull, zc, 0)
    if rem:
        pltpu.sync_copy(ones_v.at[pl.ds(0, rem)],
                        sh.at[pl.ds(pl.multiple_of(slab0 + nfull * KG, 8),
                                    rem)])

    def ob(i, _):
        for dd in range(8):
            ones_v[i, pl.ds(dd * 16, 16)] = one16
        return 0
    lax.fori_loop(0, KG, ob, 0)
    plsc.subcore_barrier()

    def grp(g, _):
        pltpu.sync_copy(
            rows_hbm.at[s, pl.ds(pl.multiple_of(g * 8 * KG, 128), 8 * KG)],
            ridx_g)
        for j in range(8):
            idx = ridx_g.at[pl.ds(j * KG, KG)]
            pltpu.sync_copy(ones_v, sh.at[idx], add=True)
        return 0
    lax.fori_loop(0, CHD // 8, grp, 0)
    plsc.subcore_barrier()

    @pl.when(c == 0)
    def _():
        pltpu.sync_copy(sh.at[pl.ds(slab0, slab)],
                        deg_hbm.at[pl.ds(slab0, slab)])


_degscale = functools.partial(
    pl.kernel,
    out_type=jax.ShapeDtypeStruct((NA_CAT, 128), F32),
    mesh=_MESH,
    scratch_types=[
        pltpu.VMEM_SHARED((NA_CAT, 128), F32),
        pltpu.VMEM((KG, 128), F32),
        pltpu.VMEM((8 * KG,), I32),
    ],
)(_degscale_body)


def _degfinish(deg2d):
    def body(d_ref, q_ref, g_ref, qi_ref):
        d = d_ref[...][:, :16]
        pos = d > 0.0
        safe = jnp.where(pos, d, 1.0)
        q_ref[...] = jnp.where(pos, jnp.sqrt(safe), 1.0)
        g_ref[...] = jnp.where(pos, -1.0 / safe, 0.0)
        qi_ref[...] = jnp.where(pos, lax.rsqrt(safe), 1.0)

    bn = NA_CAT // NT
    sds = jax.ShapeDtypeStruct((NA_CAT, 16), F32)
    spec = pl.BlockSpec((bn, 16), lambda i: (i, 0))
    ispec = pl.BlockSpec((bn, 128), lambda i: (i, 0))
    q, g, qi = pl.pallas_call(
        body, grid=(NT,), in_specs=[ispec], out_specs=(spec, spec, spec),
        out_shape=(sds, sds, sds))(deg2d)
    return q[:, 0], g[:, 0], qi[:, 0]


def _make_gather(dh, nch):
    def body(hin, pcol, out, rows, cidx, sem0, sem1):
        c = lax.axis_index("c")
        s = lax.axis_index("s")
        ep = nch * KG
        base = pl.multiple_of(c * NT * ep + s * ep, 8)

        def grp(g, _):
            goff = pl.multiple_of(g * 8, 8)
            pltpu.sync_copy(pcol.at[c, s, pl.ds(goff, 8)], cidx)
            pltpu.async_copy(hin.at[cidx.at[0]], rows.at[0], sem0)
            for p in range(4):
                i0, i1 = 2 * p, 2 * p + 1
                dst0 = pl.multiple_of(base + (g * 8 + i0) * KG, 8)
                dst1 = pl.multiple_of(base + (g * 8 + i1) * KG, 8)
                pltpu.async_copy(hin.at[cidx.at[i1]], rows.at[1], sem1)
                pltpu.make_async_copy(hin.at[cidx.at[i0]], rows.at[0],
                                      sem0).wait()
                pltpu.sync_copy(rows.at[0], out.at[pl.ds(dst0, KG)])
                if p < 3:
                    pltpu.async_copy(hin.at[cidx.at[i0 + 2]], rows.at[0], sem0)
                pltpu.make_async_copy(hin.at[cidx.at[i1]], rows.at[1],
                                      sem1).wait()
                pltpu.sync_copy(rows.at[1], out.at[pl.ds(dst1, KG)])
            return 0
        lax.fori_loop(0, nch // 8, grp, 0)

    return functools.partial(
        pl.kernel,
        out_type=jax.ShapeDtypeStruct((2 * NT * nch * KG, dh), F32),
        mesh=_MESH,
        scratch_types=[
            pltpu.VMEM((2, KG, dh), F32),
            pltpu.VMEM((8, KG), I32),
            pltpu.SemaphoreType.DMA,
            pltpu.SemaphoreType.DMA,
        ],
    )(body)


_gather21 = _make_gather(128, 8192 // (NT * KG))
_gather0 = _make_gather(128, 32768 // (NT * KG))


def _make_level(n_acc, dh, nch):
    slab = n_acc // NT
    epc = 32
    n_ep = slab // epc
    nd = dh // 16

    def body(u0, v0, crow, ccol, g2d, uall, vb,
             acc, rows, cidx, ridx, abuf, vvb, zbuf, gb2, sem0, sem1):
        c = lax.axis_index("c")
        s = lax.axis_index("s")
        slab0 = pl.multiple_of(s * slab, 8)
        zero16 = jnp.zeros((16,), F32)

        def zb(i, _):
            for d in range(nd):
                zbuf[i, pl.ds(d * 16, 16)] = zero16
            return 0
        lax.fori_loop(0, epc, zb, 0)

        def zc(i, _):
            pltpu.sync_copy(
                zbuf, acc.at[pl.ds(pl.multiple_of(slab0 + i * epc, 8), epc)])
            return 0
        lax.fori_loop(0, n_ep, zc, 0)
        pltpu.sync_copy(g2d.at[pl.ds(pl.multiple_of(s * (slab // 16), 8),
                                     slab // 16)], gb2)
        plsc.subcore_barrier()

        def ep_chunk(k, i, _):
            o = pl.multiple_of(i * epc, epc)
            rb = pl.multiple_of(slab0 + o, 8)
            pltpu.sync_copy(acc.at[pl.ds(rb, epc)], abuf)
            if k >= 2:
                if k == 2:
                    pltpu.sync_copy(u0.at[pl.ds(c * n_acc + rb, epc)], vvb)
                else:
                    src = (k - 3) * 2 * n_acc + c * n_acc + rb
                    pltpu.sync_copy(uall.at[pl.ds(src, epc)], vvb)

            def blk(j16, _):
                gv = gb2[o // 16 + j16, :]
                for jj in range(16):
                    gs = gv[jj]
                    j = j16 * 16 + jj
                    for d in range(nd):
                        a = abuf[j, pl.ds(d * 16, 16)]
                        if k == 1:
                            u = a
                        else:
                            u = 2.0 * a - vvb[j, pl.ds(d * 16, 16)]
                        abuf[j, pl.ds(d * 16, 16)] = u
                        if k < 5:
                            vvb[j, pl.ds(d * 16, 16)] = u * gs
                return 0
            lax.fori_loop(0, epc // 16, blk, 0)
            dst = (k - 1) * 2 * n_acc + c * n_acc + rb
            pltpu.sync_copy(abuf, uall.at[pl.ds(dst, epc)])
            if k < 5:
                pltpu.sync_copy(vvb, vb.at[pl.ds(c * n_acc + rb, epc)])
            pltpu.sync_copy(zbuf, acc.at[pl.ds(rb, epc)])
            return 0

        for k in range(1, 6):
            src_ref = v0 if k == 1 else vb

            def grp(g, _):
                goff = pl.multiple_of(g * 8, 8)
                pltpu.sync_copy(ccol.at[c, s, pl.ds(goff, 8)], cidx)
                pltpu.sync_copy(
                    crow.at[s, pl.ds(pl.multiple_of(g * 8 * KG, 128),
                                     8 * KG)], ridx)
                pltpu.async_copy(src_ref.at[cidx.at[0]], rows.at[0], sem0)
                for p in range(4):
                    i0, i1 = 2 * p, 2 * p + 1
                    r0 = ridx.at[pl.ds(i0 * KG, KG)]
                    r1 = ridx.at[pl.ds(i1 * KG, KG)]
                    pltpu.async_copy(src_ref.at[cidx.at[i1]], rows.at[1],
                                     sem1)
                    pltpu.make_async_copy(src_ref.at[cidx.at[i0]], rows.at[0],
                                          sem0).wait()
                    pltpu.sync_copy(rows.at[0], acc.at[r0], add=True)
                    if p < 3:
                        pltpu.async_copy(src_ref.at[cidx.at[i0 + 2]],
                                         rows.at[0], sem0)
                    pltpu.make_async_copy(src_ref.at[cidx.at[i1]], rows.at[1],
                                          sem1).wait()
                    pltpu.sync_copy(rows.at[1], acc.at[r1], add=True)
                return 0
            lax.fori_loop(0, nch // 8, grp, 0)
            plsc.subcore_barrier()
            lax.fori_loop(0, n_ep, functools.partial(ep_chunk, k), 0)
            plsc.subcore_barrier()

    return functools.partial(
        pl.kernel,
        out_type=(jax.ShapeDtypeStruct((5 * 2 * n_acc, dh), F32),
                  jax.ShapeDtypeStruct((2 * n_acc, dh), F32)),
        mesh=_MESH,
        scratch_types=[
            pltpu.VMEM_SHARED((n_acc, dh), F32),
            pltpu.VMEM((2, KG, dh), F32),
            pltpu.VMEM((8, KG), I32),
            pltpu.VMEM((8 * KG,), I32),
            pltpu.VMEM((epc, dh), F32),
            pltpu.VMEM((epc, dh), F32),
            pltpu.VMEM((epc, dh), F32),
            pltpu.VMEM((slab // 16, 16), F32),
            pltpu.SemaphoreType.DMA,
            pltpu.SemaphoreType.DMA,
        ],
    )(body)


_level2 = _make_level(2 * NA2, 128, 32768 // (NT * KG))
_level1 = _make_level(NA1, 128, 40960 // (NT * KG))
_level0 = _make_level(NA0, 128, 163840 // (NT * KG))


def _fc_pallas(x, w, b):
    def body(x_ref, w_ref, b_ref, o_ref):
        o_ref[...] = jnp.dot(x_ref[...], w_ref[...],
                             preferred_element_type=F32) + b_ref[...]

    return pl.pallas_call(
        body,
        out_shape=jax.ShapeDtypeStruct((x.shape[0], w.shape[1]), F32),
    )(x, w, b.reshape(1, -1))


def _poolcomb(g3, val3, q, gsc, n_out, n_acc, dh):
    bn = 128
    nb = pl.cdiv(n_out, bn)

    def body(g_ref, v_ref, q_ref, gs_ref, u_ref, vv_ref):
        q_ = q_ref[...]
        gs = gs_ref[...]
        for sidx in range(2):
            gblk = g_ref[sidx]
            y = (gblk[:, 0 * dh:1 * dh] * v_ref[..., 0:1]
                 + gblk[:, 1 * dh:2 * dh] * v_ref[..., 1:2]
                 + gblk[:, 2 * dh:3 * dh] * v_ref[..., 2:3])
            u = y * q_
            u_ref[sidx] = u
            vv_ref[sidx] = u * gs

    sds = jax.ShapeDtypeStruct((2, n_acc, dh), F32)
    ospec = pl.BlockSpec((2, bn, dh), lambda i: (0, i, 0))
    return pl.pallas_call(
        body,
        grid=(nb,),
        in_specs=[
            pl.BlockSpec((2, bn, 3 * dh), lambda i: (0, i, 0)),
            pl.BlockSpec((bn, 8), lambda i: (i, 0)),
            pl.BlockSpec((bn, 1), lambda i: (i, 0)),
            pl.BlockSpec((bn, 1), lambda i: (i, 0)),
        ],
        out_specs=(ospec, ospec),
        out_shape=(sds, sds),
    )(g3, val3, q, gsc)


def _prep(h, q, gsc, n_acc, dh):
    bn = 128

    def body(h_ref, q_ref, gs_ref, u_ref, vv_ref):
        q_ = q_ref[...]
        gs = gs_ref[...]
        for sidx in range(2):
            u = h_ref[sidx] * q_
            u_ref[sidx] = u
            vv_ref[sidx] = u * gs

    sds = jax.ShapeDtypeStruct((2, n_acc, dh), F32)
    ospec = pl.BlockSpec((2, bn, dh), lambda i: (0, i, 0))
    return pl.pallas_call(
        body,
        grid=(n_acc // bn,),
        in_specs=[
            pl.BlockSpec((2, bn, dh), lambda i: (0, i, 0)),
            pl.BlockSpec((bn, 1), lambda i: (i, 0)),
            pl.BlockSpec((bn, 1), lambda i: (i, 0)),
        ],
        out_specs=(ospec, ospec),
        out_shape=(sds, sds),
    )(h, q, gsc)


def _make_combine(n_acc, dh, dh_out, do_bn):
    bn = 128
    nb = n_acc // bn

    def body(u0_ref, u_ref, qi_ref, w_ref, b_ref, o_ref):
        w = w_ref[...]
        qi = qi_ref[...]
        bias = b_ref[...]
        ys = []
        for sidx in range(2):
            xcat = jnp.concatenate(
                [u0_ref[sidx]] + [u_ref[kk, sidx] for kk in range(5)],
                axis=-1)
            y = jnp.dot(xcat, w, preferred_element_type=F32)
            y = y * qi + bias
            ys.append(y)
        if do_bn:
            denom = 2.0 * dh_out
            m = (jnp.sum(ys[0], axis=1, keepdims=True)
                 + jnp.sum(ys[1], axis=1, keepdims=True)) / denom
            s2 = (jnp.sum(ys[0] * ys[0], axis=1, keepdims=True)
                  + jnp.sum(ys[1] * ys[1], axis=1, keepdims=True)) / denom
            r = lax.rsqrt(s2 - m * m + 1e-5)
            out = []
            for y in ys:
                z = (y - m) * r
                out.append(jnp.where(z > 0, z, jnp.exp(z) - 1.0))
            ys = out
        o_ref[...] = jnp.stack(ys, axis=0)

    def run(u0, uall, qi, w, b):
        return pl.pallas_call(
            body,
            grid=(nb,),
            in_specs=[
                pl.BlockSpec((2, bn, dh), lambda i: (0, i, 0)),
                pl.BlockSpec((5, 2, bn, dh), lambda i: (0, 0, i, 0)),
                pl.BlockSpec((bn, 1), lambda i: (i, 0)),
                pl.BlockSpec((6 * dh, dh_out), lambda i: (0, 0)),
                pl.BlockSpec((1, dh_out), lambda i: (0, 0)),
            ],
            out_specs=pl.BlockSpec((2, bn, dh_out), lambda i: (0, i, 0)),
            out_shape=jax.ShapeDtypeStruct((2, n_acc, dh_out), F32),
        )(u0, uall, qi, w, b)
    return run


def _poolcomb2q(g4, val3, q, gsc):
    bn = 128
    dh = 128

    def body(g_ref, v_ref, q_ref, gs_ref, u_ref, vv_ref):
        q_ = q_ref[...]
        gs = gs_ref[...]
        for qq in range(4):
            gblk = g_ref[qq]
            y = (gblk[:, 0 * dh:1 * dh] * v_ref[..., 0:1]
                 + gblk[:, 1 * dh:2 * dh] * v_ref[..., 1:2]
                 + gblk[:, 2 * dh:3 * dh] * v_ref[..., 2:3])
            u = y * q_
            u_ref[qq] = u
            vv_ref[qq] = u * gs

    sds = jax.ShapeDtypeStruct((4, NA2, dh), F32)
    ospec = pl.BlockSpec((4, bn, dh), lambda i: (0, i, 0))
    return pl.pallas_call(
        body,
        grid=(pl.cdiv(N2, bn),),
        in_specs=[
            pl.BlockSpec((4, bn, 3 * dh), lambda i: (0, i, 0)),
            pl.BlockSpec((bn, 8), lambda i: (i, 0)),
            pl.BlockSpec((bn, 1), lambda i: (i, 0)),
            pl.BlockSpec((bn, 1), lambda i: (i, 0)),
        ],
        out_specs=(ospec, ospec),
        out_shape=(sds, sds),
    )(g4, val3, q, gsc)


def _combine2q(u0, uall, qi, w, b):
    bn = 128
    dh = 128
    dho = 64

    def body(u0_ref, u_ref, qi_ref, w_ref, b_ref, o_ref):
        w_ = w_ref[...]
        qi_ = qi_ref[...]
        bias = b_ref[...]
        ys = []
        for qq in range(4):
            xcat = jnp.concatenate(
                [u0_ref[qq]] + [u_ref[kk, qq] for kk in range(5)], axis=-1)
            y = jnp.dot(xcat, w_, preferred_element_type=F32)
            ys.append(y * qi_ + bias)
        denom = 4.0 * dho
        m = sum(jnp.sum(y, axis=1, keepdims=True) for y in ys) / denom
        s2 = sum(jnp.sum(y * y, axis=1, keepdims=True) for y in ys) / denom
        r = lax.rsqrt(s2 - m * m + 1e-5)
        zs = []
        for y in ys:
            z = (y - m) * r
            zs.append(jnp.where(z > 0, z, jnp.exp(z) - 1.0))
        o_ref[...] = jnp.stack(
            [jnp.concatenate([zs[0], zs[1]], axis=-1),
             jnp.concatenate([zs[2], zs[3]], axis=-1)], axis=0)

    return pl.pallas_call(
        body,
        grid=(NA2 // bn,),
        in_specs=[
            pl.BlockSpec((4, bn, dh), lambda i: (0, i, 0)),
            pl.BlockSpec((5, 4, bn, dh), lambda i: (0, 0, i, 0)),
            pl.BlockSpec((bn, 1), lambda i: (i, 0)),
            pl.BlockSpec((6 * dh, dho), lambda i: (0, 0)),
            pl.BlockSpec((1, dho), lambda i: (0, 0)),
        ],
        out_specs=pl.BlockSpec((2, bn, dh), lambda i: (0, i, 0)),
        out_shape=jax.ShapeDtypeStruct((2, NA2, dh), F32),
    )(u0, uall, qi, w, b)


_combine1 = _make_combine(NA1, 128, 128, True)
_combine0 = _make_combine(NA0, 128, 128, True)
_combinef = _make_combine(NA0, 128, 64, False)


def _pad_cols(col, n_in_rows, e_pad):
    col = col.astype(I32)
    cols = jnp.stack([col, col + n_in_rows])
    cols = jnp.pad(cols, ((0, 0), (0, e_pad - col.shape[0])))
    return cols.reshape(2, NT, -1, KG)


def _pad_rows(row, n_scratch, e_pad):
    row = jnp.pad(row.astype(I32), (0, e_pad - row.shape[0]),
                  constant_values=n_scratch)
    return row.reshape(NT, -1)


def _nvec(a, n_acc):
    return jnp.pad(a, (0, n_acc - a.shape[0])).reshape(-1, 1)


def kernel(x, edge_index0, edge_index1, edge_index2, ut0_row, ut0_col,
           ut0_val, ut1_row, ut1_col, ut1_val, ut2_row, ut2_col, ut2_val,
           fc_w, fc_b, W0, b0, W1, b1, W2, b2, Wr, br):
    rows_cat = jnp.concatenate([
        edge_index0[0].astype(I32),
        edge_index1[0].astype(I32) + N0,
        edge_index2[0].astype(I32) + N0 + N1,
    ])
    rows_cat = jnp.pad(rows_cat, (0, E_CAT_PAD - E_CAT), constant_values=N_CAT)
    q_cat, g_cat, qi_cat = _degfinish(_degscale(rows_cat.reshape(NT, -1)))
    q0, g0, qi0 = (_nvec(q_cat[:N0], NA0), _nvec(g_cat[:N0], NA0),
                   _nvec(qi_cat[:N0], NA0))
    q1, g1, qi1 = (_nvec(q_cat[N0:N0 + N1], NA1),
                   _nvec(g_cat[N0:N0 + N1], NA1),
                   _nvec(qi_cat[N0:N0 + N1], NA1))
    q2, g2, qi2 = (_nvec(q_cat[N0 + N1:N_CAT], NA2),
                   _nvec(g_cat[N0 + N1:N_CAT], NA2),
                   _nvec(qi_cat[N0 + N1:N_CAT], NA2))
    g0d = g0.reshape(-1, 16)
    g1d = g1.reshape(-1, 16)
    g2vd = jnp.concatenate([g2, g2]).reshape(-1, 16)

    h = _fc_pallas(x, fc_w, fc_b)
    h3 = h.reshape(4, 4, N3, 32).transpose(0, 2, 1, 3).reshape(4 * N3, 128)

    eye8 = jnp.eye(8, dtype=F32)

    def wbd(W):
        return jnp.concatenate([jnp.kron(eye8, W[k]) for k in range(6)], 0)
    bt0 = jnp.tile(b0, 8).reshape(1, -1)
    bt1 = jnp.tile(b1, 8).reshape(1, -1)
    bt2 = jnp.tile(b2, 8).reshape(1, -1)
    btr = jnp.tile(jnp.pad(br, (0, 5)), 8).reshape(1, -1)
    Wrp = jnp.pad(Wr, ((0, 0), (0, 0), (0, 5)))

    def pool_stage(gathered, val, q, gsc, n_out, n_acc, dh):
        g3 = gathered.reshape(2, -1, dh)[:, :3 * n_out]
        g3 = g3.reshape(2, n_out, 3 * dh)
        val3 = jnp.pad(val.astype(F32).reshape(n_out, 3), ((0, 0), (0, 5)))
        return _poolcomb(g3, val3, q, gsc, n_out, n_acc, dh)

    def run_level(level, combine, u0, v0, ei, n_out, n_acc, gd, qi, W, bt,
                  e_pad):
        cr = _pad_rows(ei[0], n_out, e_pad)
        cc = _pad_cols(ei[1], n_acc, e_pad)
        uall, _ = level(u0.reshape(2 * n_acc, -1), v0.reshape(2 * n_acc, -1),
                        cr, cc, gd)
        dh = u0.shape[-1]
        return combine(u0, uall.reshape(5, 2, n_acc, dh), qi, wbd(W), bt)

    pc2 = ut2_col.astype(I32)
    gth = _gather21(h3, _pad_cols(jnp.concatenate([pc2, pc2 + N3]),
                                  2 * N3, 8192))
    G2 = gth.reshape(2, 8192, 128)
    qa = G2[:, 0:3 * N2].reshape(2, N2, 384)
    qb = G2[:, 3 * N2:6 * N2].reshape(2, N2, 384)
    g4 = jnp.stack([qa[0], qb[0], qa[1], qb[1]])
    val3 = jnp.pad(ut2_val.astype(F32).reshape(N2, 3), ((0, 0), (0, 5)))
    u0q, v0q = _poolcomb2q(g4, val3, q2, g2)
    r2, c2 = edge_index2[0].astype(I32), edge_index2[1].astype(I32)
    cr = _pad_rows(jnp.concatenate([r2, r2 + NA2]), N2, 32768)
    cc = _pad_cols(jnp.concatenate([c2, c2 + NA2]), 2 * NA2, 32768)
    uall2, _ = _level2(u0q.reshape(4 * NA2, 128), v0q.reshape(4 * NA2, 128),
                       cr, cc, g2vd)
    wbd2 = jnp.concatenate(
        [jnp.kron(jnp.eye(4, dtype=F32), W0[k]) for k in range(6)], 0)
    h2 = _combine2q(u0q, uall2.reshape(5, 4, NA2, 128), qi2, wbd2,
                    jnp.tile(b2, 4).reshape(1, -1))
    gth = _gather21(h2.reshape(2 * NA2, 128), _pad_cols(ut1_col, NA2, 8192))
    u0, v0 = pool_stage(gth, ut1_val, q1, g1, N1, NA1, 128)
    h1 = run_level(_level1, _combine1, u0, v0, edge_index1, N1, NA1, g1d,
                   qi1, W1, bt1, 40960)
    gth = _gather0(h1.reshape(2 * NA1, 128), _pad_cols(ut0_col, NA1, 32768))
    u0, v0 = pool_stage(gth, ut0_val, q0, g0, N0, NA0, 128)
    h0 = run_level(_level0, _combine0, u0, v0, edge_index0, N0, NA0, g0d,
                   qi0, W2, bt0, 163840)
    u0, v0 = _prep(h0, q0, g0, NA0, 128)
    y = run_level(_level0, _combinef, u0, v0, edge_index0, N0, NA0, g0d,
                  qi0, Wrp, btr, 163840)

    out = (y[:, :N0, :].reshape(2, N0, 8, 8).transpose(0, 2, 1, 3)
           .reshape(NBATCH, N0, 8)[:, :, :3])
    return out
